# baseline (speedup 1.0000x reference)
import functools

import jax
import jax.numpy as jnp
from jax import lax
from jax.experimental import pallas as pl
from jax.experimental.pallas import tpu as pltpu
from jax.experimental.pallas import tpu_sc as plsc

N = 10000
D = 128
NC = 2
NS = 16
NW = NC * NS
K = 64
NBUF = 4
ROWS_PER_TILE = 632
N_PAD = NS * ROWS_PER_TILE
_PUB_SIZES = [64] * 9 + [56]
DUMMY_ROW = N


def _sc_agg_body(ch_per_tile, h_hbm, src_hbm, dst_hbm, *refs):
    out_hbm, sidx_all, dbuf, rows, isem, acc, dsem, gsem = refs
    c = lax.axis_index("c")
    s = lax.axis_index("s")
    wid = c * NS + s
    ept = ch_per_tile * K

    idx_src_cp = pltpu.async_copy(
        src_hbm.at[pl.ds(wid * ept, ept)], sidx_all, isem)

    def dst_cp(ch, b):
        off = pl.multiple_of(wid * ept + ch * K, K)
        return pltpu.async_copy(dst_hbm.at[pl.ds(off, K)], dbuf[b], dsem[b])

    def gather_cp(ch, b):
        off = pl.multiple_of(ch * K, K)
        return pltpu.async_copy(
            h_hbm.at[sidx_all.at[pl.ds(off, K)]], rows[b], gsem[b])

    zv = jnp.zeros((16,), jnp.float32)

    @pl.loop(0, K)
    def _(i):
        for j in range(D // 16):
            rows[0][i, pl.ds(j * 16, 16)] = zv

    r0 = s * ROWS_PER_TILE
    for sz in _PUB_SIZES:
        pltpu.sync_copy(rows[0].at[pl.ds(0, sz)], acc.at[pl.ds(r0, sz)])
        r0 += sz

    idx_src_cp.wait()
    plsc.subcore_barrier()

    for b in range(NBUF):
        dst_cp(b, b)
        gather_cp(b, b)

    @pl.loop(0, ch_per_tile, step=NBUF)
    def _(i):
        for b in range(NBUF):
            ch = i + b
            gather_cp(ch, b).wait()
            pltpu.make_async_copy(
                dst_hbm.at[pl.ds(0, K)], dbuf[b], dsem[b]).wait()
            pltpu.sync_copy(rows[b], acc.at[dbuf[b]], add=True)
            nxt = ch + NBUF

            @pl.when(nxt < ch_per_tile)
            def _():
                dst_cp(nxt, b)
                gather_cp(nxt, b)

    plsc.subcore_barrier()

    r0 = s * ROWS_PER_TILE
    for sz in _PUB_SIZES:
        pltpu.sync_copy(acc.at[pl.ds(r0, sz)], rows[0].at[pl.ds(0, sz)])
        pltpu.sync_copy(rows[0].at[pl.ds(0, sz)], out_hbm.at[c, pl.ds(r0, sz)])
        r0 += sz


def _sc_deg_body(ch_per_tile, dst_hbm, deg_hbm, didx, ones_b, zbuf,
                 degacc, sem):
    c = lax.axis_index("c")
    s = lax.axis_index("s")
    wid = c * NS + s
    ept = ch_per_tile * K

    zv = jnp.zeros((16,), jnp.float32)
    ov = jnp.ones((16,), jnp.float32)

    @pl.loop(0, K)
    def _(i):
        zbuf[i, pl.ds(0, 16)] = zv
        ones_b[i, pl.ds(0, 16)] = ov

    r0 = s * ROWS_PER_TILE
    for sz in _PUB_SIZES:
        pltpu.sync_copy(zbuf.at[pl.ds(0, sz)], degacc.at[pl.ds(r0, sz)])
        r0 += sz

    plsc.subcore_barrier()

    @pl.loop(0, ch_per_tile)
    def _(i):
        off = pl.multiple_of(wid * ept + i * K, K)
        pltpu.sync_copy(dst_hbm.at[pl.ds(off, K)], didx)
        pltpu.sync_copy(ones_b, degacc.at[didx], add=True)

    plsc.subcore_barrier()

    r0 = s * ROWS_PER_TILE
    for sz in _PUB_SIZES:
        pltpu.sync_copy(degacc.at[pl.ds(r0, sz)], zbuf.at[pl.ds(0, sz)])
        pltpu.sync_copy(zbuf.at[pl.ds(0, sz)], deg_hbm.at[c, pl.ds(r0, sz)])
        r0 += sz


def _make_sc_agg(ch_per_tile):
    mesh = plsc.VectorSubcoreMesh(core_axis_name="c", subcore_axis_name="s")
    return pl.kernel(
        functools.partial(_sc_agg_body, ch_per_tile),
        out_type=jax.ShapeDtypeStruct((NC, N_PAD, D), jnp.float32),
        mesh=mesh,
        scratch_types=(
            pltpu.VMEM((ch_per_tile * K,), jnp.int32),
            tuple(pltpu.VMEM((K,), jnp.int32) for _ in range(NBUF)),
            tuple(pltpu.VMEM((K, D), jnp.float32) for _ in range(NBUF)),
            pltpu.SemaphoreType.DMA,
            pltpu.VMEM_SHARED((N_PAD, D), jnp.float32),
            tuple(pltpu.SemaphoreType.DMA for _ in range(NBUF)),
            tuple(pltpu.SemaphoreType.DMA for _ in range(NBUF)),
        ),
    )


def _make_sc_deg(ch_per_tile):
    mesh = plsc.VectorSubcoreMesh(core_axis_name="c", subcore_axis_name="s")
    return pl.kernel(
        functools.partial(_sc_deg_body, ch_per_tile),
        out_type=jax.ShapeDtypeStruct((NC, N_PAD, 16), jnp.float32),
        mesh=mesh,
        scratch_types=(
            pltpu.VMEM((K,), jnp.int32),
            pltpu.VMEM((K, 16), jnp.float32),
            pltpu.VMEM((K, 16), jnp.float32),
            pltpu.VMEM_SHARED((N_PAD, 16), jnp.float32),
            pltpu.SemaphoreType.DMA,
        ),
    )


def _tc_mean_layer_body(p0, p1, d0, d1, h, wl, bl, wr, out):
    deg = d0[:, 0:1] + d1[:, 0:1]
    inv = 1.0 / jnp.maximum(deg, 1.0)
    agg = (p0[...] + p1[...]) * inv
    y = (jnp.dot(agg, wl[...], preferred_element_type=jnp.float32)
         + bl[...]
         + jnp.dot(h[...], wr[...], preferred_element_type=jnp.float32))
    out[...] = jnp.maximum(y, 0.0)


def _tc_final_layer_body(p0, p1, h, wl, bl, wr, wlin, blin, out):
    agg = p0[...] + p1[...]
    y = (jnp.dot(agg, wl[...], preferred_element_type=jnp.float32)
         + bl[...]
         + jnp.dot(h[...], wr[...], preferred_element_type=jnp.float32))
    hh = jnp.maximum(y, 0.0)
    o = jnp.dot(hh, wlin[...], preferred_element_type=jnp.float32) + blin[...]
    e = jnp.exp(o - jnp.max(o, axis=1, keepdims=True))
    out[...] = e / jnp.sum(e, axis=1, keepdims=True)


_BM = 1264


def _row_spec(width):
    return pl.BlockSpec((_BM, width), lambda i: (i, 0))


def _full_spec(r, ccol):
    return pl.BlockSpec((r, ccol), lambda i: (0, 0))


def _tc_mean_layer(p0, p1, d0, d1, h, wl, bl, wr):
    return pl.pallas_call(
        _tc_mean_layer_body,
        grid=(N_PAD // _BM,),
        in_specs=[
            _row_spec(D), _row_spec(D), _row_spec(16), _row_spec(16),
            _row_spec(D), _full_spec(D, D), _full_spec(1, D), _full_spec(D, D),
        ],
        out_specs=_row_spec(D),
        out_shape=jax.ShapeDtypeStruct((N_PAD, D), jnp.float32),
    )(p0, p1, d0, d1, h, wl, bl.reshape(1, D), wr)


def _tc_final_layer(p0, p1, h, wl, bl, wr, wlin, blin):
    return pl.pallas_call(
        _tc_final_layer_body,
        grid=(N_PAD // _BM,),
        in_specs=[
            _row_spec(D), _row_spec(D), _row_spec(D),
            _full_spec(D, D), _full_spec(1, D), _full_spec(D, D),
            _full_spec(D, 1), _full_spec(1, 1),
        ],
        out_specs=_row_spec(1),
        out_shape=jax.ShapeDtypeStruct((N_PAD, 1), jnp.float32),
    )(p0, p1, h, wl, bl.reshape(1, D), wr, wlin, blin.reshape(1, 1))


def kernel(x, edge_index, W1l, b1l, W1r, W2l, b2l, W2r, W3l, b3l, W3r,
           Wlin, blin):
    e = edge_index.shape[1]
    ch_per_tile = -(-e // (NW * K))
    ch_per_tile = -(-ch_per_tile // NBUF) * NBUF
    e_pad = ch_per_tile * NW * K
    src = edge_index[0].astype(jnp.int32)
    dst = edge_index[1].astype(jnp.int32)
    pad = e_pad - e
    if pad:
        src = jnp.concatenate([src, jnp.zeros((pad,), jnp.int32)])
        dst = jnp.concatenate([dst, jnp.full((pad,), DUMMY_ROW, jnp.int32)])
    xp = jnp.concatenate([x, jnp.zeros((N_PAD - N, D), x.dtype)])

    sc_agg = _make_sc_agg(ch_per_tile)
    sc_deg = _make_sc_deg(ch_per_tile)

    def _one(r):
        return r[0] if isinstance(r, (tuple, list)) else r

    dp = _one(sc_deg(dst))
    p = _one(sc_agg(xp, src, dst))
    h1 = _tc_mean_layer(p[0], p[1], dp[0], dp[1], xp, W1l, b1l, W1r)
    p = _one(sc_agg(h1, src, dst))
    h2 = _tc_mean_layer(p[0], p[1], dp[0], dp[1], h1, W2l, b2l, W2r)
    p = _one(sc_agg(h2, src, dst))
    out = _tc_final_layer(p[0], p[1], h2, W3l, b3l, W3r, Wlin, blin)
    return out[:N]


# speedup vs baseline: 1.8083x; 1.8083x over previous
import functools

import jax
import jax.numpy as jnp
from jax import lax
from jax.experimental import pallas as pl
from jax.experimental.pallas import tpu as pltpu
from jax.experimental.pallas import tpu_sc as plsc

N = 10000
D = 128
NC = 2
NS = 16
NW = NC * NS
K = 64
NBUF = 4
ROWS_PER_TILE = 632
N_PAD = NS * ROWS_PER_TILE
_PUB_SIZES = [64] * 9 + [56]
DUMMY_ROW = N


def _sc_agg_body(ch_per_tile, h_hbm, src_hbm, dst_hbm, *refs):
    out_hbm, sidx_all, dbuf, rows, isem, acc, dsem, gsem = refs
    c = lax.axis_index("c")
    s = lax.axis_index("s")
    wid = c * NS + s
    ept = ch_per_tile * K

    idx_src_cp = pltpu.async_copy(
        src_hbm.at[pl.ds(wid * ept, ept)], sidx_all, isem)

    def dst_cp(ch, b):
        off = pl.multiple_of(wid * ept + ch * K, K)
        return pltpu.async_copy(dst_hbm.at[pl.ds(off, K)], dbuf[b], dsem[b])

    def gather_cp(ch, b):
        off = pl.multiple_of(ch * K, K)
        return pltpu.async_copy(
            h_hbm.at[sidx_all.at[pl.ds(off, K)]], rows[b], gsem[b])

    def gather_wait(b):
        pltpu.make_async_copy(
            h_hbm.at[sidx_all.at[pl.ds(0, K)]], rows[b], gsem[b]).wait()

    zv = jnp.zeros((16,), jnp.float32)

    @pl.loop(0, K)
    def _(i):
        for j in range(D // 16):
            rows[0][i, pl.ds(j * 16, 16)] = zv

    r0 = s * ROWS_PER_TILE
    for sz in _PUB_SIZES:
        pltpu.sync_copy(rows[0].at[pl.ds(0, sz)], acc.at[pl.ds(r0, sz)])
        r0 += sz

    idx_src_cp.wait()
    plsc.subcore_barrier()

    for b in range(NBUF):
        dst_cp(b, b)
        gather_cp(b, b)

    @pl.loop(0, ch_per_tile, step=NBUF)
    def _(i):
        for b in range(NBUF):
            ch = i + b
            gather_wait(b)
            pltpu.make_async_copy(
                dst_hbm.at[pl.ds(0, K)], dbuf[b], dsem[b]).wait()
            pltpu.sync_copy(rows[b], acc.at[dbuf[b]], add=True)
            nxt = ch + NBUF

            @pl.when(nxt < ch_per_tile)
            def _():
                dst_cp(nxt, b)
                gather_cp(nxt, b)

    plsc.subcore_barrier()

    r0 = s * ROWS_PER_TILE
    for sz in _PUB_SIZES:
        pltpu.sync_copy(acc.at[pl.ds(r0, sz)], rows[0].at[pl.ds(0, sz)])
        pltpu.sync_copy(rows[0].at[pl.ds(0, sz)], out_hbm.at[c, pl.ds(r0, sz)])
        r0 += sz


def _sc_deg_body(ch_per_tile, dst_hbm, deg_hbm, didx, ones_b, zbuf,
                 degacc, sem):
    c = lax.axis_index("c")
    s = lax.axis_index("s")
    wid = c * NS + s
    ept = ch_per_tile * K

    zv = jnp.zeros((16,), jnp.float32)
    ov = jnp.ones((16,), jnp.float32)

    @pl.loop(0, K)
    def _(i):
        zbuf[i, pl.ds(0, 16)] = zv
        ones_b[i, pl.ds(0, 16)] = ov

    r0 = s * ROWS_PER_TILE
    for sz in _PUB_SIZES:
        pltpu.sync_copy(zbuf.at[pl.ds(0, sz)], degacc.at[pl.ds(r0, sz)])
        r0 += sz

    plsc.subcore_barrier()

    @pl.loop(0, ch_per_tile)
    def _(i):
        off = pl.multiple_of(wid * ept + i * K, K)
        pltpu.sync_copy(dst_hbm.at[pl.ds(off, K)], didx)
        pltpu.sync_copy(ones_b, degacc.at[didx], add=True)

    plsc.subcore_barrier()

    r0 = s * ROWS_PER_TILE
    for sz in _PUB_SIZES:
        pltpu.sync_copy(degacc.at[pl.ds(r0, sz)], zbuf.at[pl.ds(0, sz)])
        pltpu.sync_copy(zbuf.at[pl.ds(0, sz)], deg_hbm.at[c, pl.ds(r0, sz)])
        r0 += sz


def _make_sc_agg(ch_per_tile):
    mesh = plsc.VectorSubcoreMesh(core_axis_name="c", subcore_axis_name="s")
    return pl.kernel(
        functools.partial(_sc_agg_body, ch_per_tile),
        out_type=jax.ShapeDtypeStruct((NC, N_PAD, D), jnp.float32),
        mesh=mesh,
        scratch_types=(
            pltpu.VMEM((ch_per_tile * K,), jnp.int32),
            tuple(pltpu.VMEM((K,), jnp.int32) for _ in range(NBUF)),
            tuple(pltpu.VMEM((K, D), jnp.float32) for _ in range(NBUF)),
            pltpu.SemaphoreType.DMA,
            pltpu.VMEM_SHARED((N_PAD, D), jnp.float32),
            tuple(pltpu.SemaphoreType.DMA for _ in range(NBUF)),
            tuple(pltpu.SemaphoreType.DMA for _ in range(NBUF)),
        ),
    )


def _make_sc_deg(ch_per_tile):
    mesh = plsc.VectorSubcoreMesh(core_axis_name="c", subcore_axis_name="s")
    return pl.kernel(
        functools.partial(_sc_deg_body, ch_per_tile),
        out_type=jax.ShapeDtypeStruct((NC, N_PAD, 16), jnp.float32),
        mesh=mesh,
        scratch_types=(
            pltpu.VMEM((K,), jnp.int32),
            pltpu.VMEM((K, 16), jnp.float32),
            pltpu.VMEM((K, 16), jnp.float32),
            pltpu.VMEM_SHARED((N_PAD, 16), jnp.float32),
            pltpu.SemaphoreType.DMA,
        ),
    )


def _tc_mean_layer_body(p0, p1, d0, d1, h, wl, bl, wr, out):
    deg = d0[:, 0:1] + d1[:, 0:1]
    inv = 1.0 / jnp.maximum(deg, 1.0)
    agg = (p0[...] + p1[...]) * inv
    y = (jnp.dot(agg, wl[...], preferred_element_type=jnp.float32)
         + bl[...]
         + jnp.dot(h[...], wr[...], preferred_element_type=jnp.float32))
    out[...] = jnp.maximum(y, 0.0)


def _tc_final_layer_body(p0, p1, h, wl, bl, wr, wlin, blin, out):
    agg = p0[...] + p1[...]
    y = (jnp.dot(agg, wl[...], preferred_element_type=jnp.float32)
         + bl[...]
         + jnp.dot(h[...], wr[...], preferred_element_type=jnp.float32))
    hh = jnp.maximum(y, 0.0)
    o = jnp.dot(hh, wlin[...], preferred_element_type=jnp.float32) + blin[...]
    e = jnp.exp(o - jnp.max(o, axis=1, keepdims=True))
    out[...] = e / jnp.sum(e, axis=1, keepdims=True)


_BM = 1264


def _row_spec(width):
    return pl.BlockSpec((_BM, width), lambda i: (i, 0))


def _full_spec(r, ccol):
    return pl.BlockSpec((r, ccol), lambda i: (0, 0))


def _tc_mean_layer(p0, p1, d0, d1, h, wl, bl, wr):
    return pl.pallas_call(
        _tc_mean_layer_body,
        grid=(N_PAD // _BM,),
        in_specs=[
            _row_spec(D), _row_spec(D), _row_spec(16), _row_spec(16),
            _row_spec(D), _full_spec(D, D), _full_spec(1, D), _full_spec(D, D),
        ],
        out_specs=_row_spec(D),
        out_shape=jax.ShapeDtypeStruct((N_PAD, D), jnp.float32),
    )(p0, p1, d0, d1, h, wl, bl.reshape(1, D), wr)


def _tc_final_layer(p0, p1, h, wl, bl, wr, wlin, blin):
    return pl.pallas_call(
        _tc_final_layer_body,
        grid=(N_PAD // _BM,),
        in_specs=[
            _row_spec(D), _row_spec(D), _row_spec(D),
            _full_spec(D, D), _full_spec(1, D), _full_spec(D, D),
            _full_spec(D, 1), _full_spec(1, 1),
        ],
        out_specs=_row_spec(1),
        out_shape=jax.ShapeDtypeStruct((N_PAD, 1), jnp.float32),
    )(p0, p1, h, wl, bl.reshape(1, D), wr, wlin, blin.reshape(1, 1))


def kernel(x, edge_index, W1l, b1l, W1r, W2l, b2l, W2r, W3l, b3l, W3r,
           Wlin, blin):
    e = edge_index.shape[1]
    ch_per_tile = -(-e // (NW * K))
    ch_per_tile = -(-ch_per_tile // NBUF) * NBUF
    e_pad = ch_per_tile * NW * K
    src = edge_index[0].astype(jnp.int32)
    dst = edge_index[1].astype(jnp.int32)
    pad = e_pad - e
    if pad:
        src = jnp.concatenate([src, jnp.zeros((pad,), jnp.int32)])
        dst = jnp.concatenate([dst, jnp.full((pad,), DUMMY_ROW, jnp.int32)])
    xp = jnp.concatenate([x, jnp.zeros((N_PAD - N, D), x.dtype)])

    sc_agg = _make_sc_agg(ch_per_tile)
    sc_deg = _make_sc_deg(ch_per_tile)

    def _one(r):
        return r[0] if isinstance(r, (tuple, list)) else r

    dp = _one(sc_deg(dst))
    p = _one(sc_agg(xp, src, dst))
    h1 = _tc_mean_layer(p[0], p[1], dp[0], dp[1], xp, W1l, b1l, W1r)
    p = _one(sc_agg(h1, src, dst))
    h2 = _tc_mean_layer(p[0], p[1], dp[0], dp[1], h1, W2l, b2l, W2r)
    p = _one(sc_agg(h2, src, dst))
    out = _tc_final_layer(p[0], p[1], h2, W3l, b3l, W3r, Wlin, blin)
    return out[:N]


# speedup vs baseline: 1.8521x; 1.0242x over previous
import functools

import jax
import jax.numpy as jnp
from jax import lax
from jax.experimental import pallas as pl
from jax.experimental.pallas import tpu as pltpu
from jax.experimental.pallas import tpu_sc as plsc

N = 10000
D = 128
NC = 2
NS = 16
NW = NC * NS
K = 128
NBUF = 2
ROWS_PER_TILE = 632
N_PAD = NS * ROWS_PER_TILE
_PUB_SIZES = [128] * 4 + [120]
DUMMY_ROW = N


def _sc_agg_body(ch_per_tile, h_hbm, src_hbm, dst_hbm, *refs):
    out_hbm, sidx_all, dbuf, rows, isem, acc, dsem, gsem = refs
    c = lax.axis_index("c")
    s = lax.axis_index("s")
    wid = c * NS + s
    ept = ch_per_tile * K

    idx_src_cp = pltpu.async_copy(
        src_hbm.at[pl.ds(wid * ept, ept)], sidx_all, isem)

    def dst_cp(ch, b):
        off = pl.multiple_of(wid * ept + ch * K, K)
        return pltpu.async_copy(dst_hbm.at[pl.ds(off, K)], dbuf[b], dsem[b])

    def gather_cp(ch, b):
        off = pl.multiple_of(ch * K, K)
        return pltpu.async_copy(
            h_hbm.at[sidx_all.at[pl.ds(off, K)]], rows[b], gsem[b])

    def gather_wait(b):
        pltpu.make_async_copy(
            h_hbm.at[sidx_all.at[pl.ds(0, K)]], rows[b], gsem[b]).wait()

    zv = jnp.zeros((16,), jnp.float32)

    @pl.loop(0, K)
    def _(i):
        for j in range(D // 16):
            rows[0][i, pl.ds(j * 16, 16)] = zv

    r0 = s * ROWS_PER_TILE
    for sz in _PUB_SIZES:
        pltpu.sync_copy(rows[0].at[pl.ds(0, sz)], acc.at[pl.ds(r0, sz)])
        r0 += sz

    idx_src_cp.wait()
    plsc.subcore_barrier()

    for b in range(NBUF):
        dst_cp(b, b)
        gather_cp(b, b)

    @pl.loop(0, ch_per_tile, step=NBUF)
    def _(i):
        for b in range(NBUF):
            ch = i + b
            gather_wait(b)
            pltpu.make_async_copy(
                dst_hbm.at[pl.ds(0, K)], dbuf[b], dsem[b]).wait()
            pltpu.sync_copy(rows[b], acc.at[dbuf[b]], add=True)
            nxt = ch + NBUF

            @pl.when(nxt < ch_per_tile)
            def _():
                dst_cp(nxt, b)
                gather_cp(nxt, b)

    plsc.subcore_barrier()

    r0 = s * ROWS_PER_TILE
    for sz in _PUB_SIZES:
        pltpu.sync_copy(acc.at[pl.ds(r0, sz)], rows[0].at[pl.ds(0, sz)])
        pltpu.sync_copy(rows[0].at[pl.ds(0, sz)], out_hbm.at[c, pl.ds(r0, sz)])
        r0 += sz


def _sc_deg_body(ch_per_tile, dst_hbm, deg_hbm, didx, ones_b, zbuf,
                 degacc, sem):
    c = lax.axis_index("c")
    s = lax.axis_index("s")
    wid = c * NS + s
    ept = ch_per_tile * K

    zv = jnp.zeros((16,), jnp.float32)
    ov = jnp.ones((16,), jnp.float32)

    @pl.loop(0, K)
    def _(i):
        zbuf[i, pl.ds(0, 16)] = zv
        ones_b[i, pl.ds(0, 16)] = ov

    r0 = s * ROWS_PER_TILE
    for sz in _PUB_SIZES:
        pltpu.sync_copy(zbuf.at[pl.ds(0, sz)], degacc.at[pl.ds(r0, sz)])
        r0 += sz

    plsc.subcore_barrier()

    @pl.loop(0, ch_per_tile)
    def _(i):
        off = pl.multiple_of(wid * ept + i * K, K)
        pltpu.sync_copy(dst_hbm.at[pl.ds(off, K)], didx)
        pltpu.sync_copy(ones_b, degacc.at[didx], add=True)

    plsc.subcore_barrier()

    r0 = s * ROWS_PER_TILE
    for sz in _PUB_SIZES:
        pltpu.sync_copy(degacc.at[pl.ds(r0, sz)], zbuf.at[pl.ds(0, sz)])
        pltpu.sync_copy(zbuf.at[pl.ds(0, sz)], deg_hbm.at[c, pl.ds(r0, sz)])
        r0 += sz


def _make_sc_agg(ch_per_tile):
    mesh = plsc.VectorSubcoreMesh(core_axis_name="c", subcore_axis_name="s")
    return pl.kernel(
        functools.partial(_sc_agg_body, ch_per_tile),
        out_type=jax.ShapeDtypeStruct((NC, N_PAD, D), jnp.float32),
        mesh=mesh,
        scratch_types=(
            pltpu.VMEM((ch_per_tile * K,), jnp.int32),
            tuple(pltpu.VMEM((K,), jnp.int32) for _ in range(NBUF)),
            tuple(pltpu.VMEM((K, D), jnp.float32) for _ in range(NBUF)),
            pltpu.SemaphoreType.DMA,
            pltpu.VMEM_SHARED((N_PAD, D), jnp.float32),
            tuple(pltpu.SemaphoreType.DMA for _ in range(NBUF)),
            tuple(pltpu.SemaphoreType.DMA for _ in range(NBUF)),
        ),
    )


def _make_sc_deg(ch_per_tile):
    mesh = plsc.VectorSubcoreMesh(core_axis_name="c", subcore_axis_name="s")
    return pl.kernel(
        functools.partial(_sc_deg_body, ch_per_tile),
        out_type=jax.ShapeDtypeStruct((NC, N_PAD, 16), jnp.float32),
        mesh=mesh,
        scratch_types=(
            pltpu.VMEM((K,), jnp.int32),
            pltpu.VMEM((K, 16), jnp.float32),
            pltpu.VMEM((K, 16), jnp.float32),
            pltpu.VMEM_SHARED((N_PAD, 16), jnp.float32),
            pltpu.SemaphoreType.DMA,
        ),
    )


def _tc_mean_layer_body(p0, p1, d0, d1, h, wl, bl, wr, out):
    deg = d0[:, 0:1] + d1[:, 0:1]
    inv = 1.0 / jnp.maximum(deg, 1.0)
    agg = (p0[...] + p1[...]) * inv
    y = (jnp.dot(agg, wl[...], preferred_element_type=jnp.float32)
         + bl[...]
         + jnp.dot(h[...], wr[...], preferred_element_type=jnp.float32))
    out[...] = jnp.maximum(y, 0.0)


def _tc_final_layer_body(p0, p1, h, wl, bl, wr, wlin, blin, out):
    agg = p0[...] + p1[...]
    y = (jnp.dot(agg, wl[...], preferred_element_type=jnp.float32)
         + bl[...]
         + jnp.dot(h[...], wr[...], preferred_element_type=jnp.float32))
    hh = jnp.maximum(y, 0.0)
    o = jnp.dot(hh, wlin[...], preferred_element_type=jnp.float32) + blin[...]
    e = jnp.exp(o - jnp.max(o, axis=1, keepdims=True))
    out[...] = e / jnp.sum(e, axis=1, keepdims=True)


_BM = 1264


def _row_spec(width):
    return pl.BlockSpec((_BM, width), lambda i: (i, 0))


def _full_spec(r, ccol):
    return pl.BlockSpec((r, ccol), lambda i: (0, 0))


def _tc_mean_layer(p0, p1, d0, d1, h, wl, bl, wr):
    return pl.pallas_call(
        _tc_mean_layer_body,
        grid=(N_PAD // _BM,),
        in_specs=[
            _row_spec(D), _row_spec(D), _row_spec(16), _row_spec(16),
            _row_spec(D), _full_spec(D, D), _full_spec(1, D), _full_spec(D, D),
        ],
        out_specs=_row_spec(D),
        out_shape=jax.ShapeDtypeStruct((N_PAD, D), jnp.float32),
    )(p0, p1, d0, d1, h, wl, bl.reshape(1, D), wr)


def _tc_final_layer(p0, p1, h, wl, bl, wr, wlin, blin):
    return pl.pallas_call(
        _tc_final_layer_body,
        grid=(N_PAD // _BM,),
        in_specs=[
            _row_spec(D), _row_spec(D), _row_spec(D),
            _full_spec(D, D), _full_spec(1, D), _full_spec(D, D),
            _full_spec(D, 1), _full_spec(1, 1),
        ],
        out_specs=_row_spec(1),
        out_shape=jax.ShapeDtypeStruct((N_PAD, 1), jnp.float32),
    )(p0, p1, h, wl, bl.reshape(1, D), wr, wlin, blin.reshape(1, 1))


def kernel(x, edge_index, W1l, b1l, W1r, W2l, b2l, W2r, W3l, b3l, W3r,
           Wlin, blin):
    e = edge_index.shape[1]
    ch_per_tile = -(-e // (NW * K))
    ch_per_tile = -(-ch_per_tile // NBUF) * NBUF
    e_pad = ch_per_tile * NW * K
    src = edge_index[0].astype(jnp.int32)
    dst = edge_index[1].astype(jnp.int32)
    pad = e_pad - e
    if pad:
        src = jnp.concatenate([src, jnp.zeros((pad,), jnp.int32)])
        dst = jnp.concatenate([dst, jnp.full((pad,), DUMMY_ROW, jnp.int32)])
    xp = jnp.concatenate([x, jnp.zeros((N_PAD - N, D), x.dtype)])

    sc_agg = _make_sc_agg(ch_per_tile)
    sc_deg = _make_sc_deg(ch_per_tile)

    def _one(r):
        return r[0] if isinstance(r, (tuple, list)) else r

    dp = _one(sc_deg(dst))
    p = _one(sc_agg(xp, src, dst))
    h1 = _tc_mean_layer(p[0], p[1], dp[0], dp[1], xp, W1l, b1l, W1r)
    p = _one(sc_agg(h1, src, dst))
    h2 = _tc_mean_layer(p[0], p[1], dp[0], dp[1], h1, W2l, b2l, W2r)
    p = _one(sc_agg(h2, src, dst))
    out = _tc_final_layer(p[0], p[1], h2, W3l, b3l, W3r, Wlin, blin)
    return out[:N]


# speedup vs baseline: 6.0776x; 3.2815x over previous
import functools

import jax
import jax.numpy as jnp
from jax import lax
from jax.experimental import pallas as pl
from jax.experimental.pallas import tpu as pltpu
from jax.experimental.pallas import tpu_sc as plsc

N = 10000
D = 128
NC = 2
NS = 16
NW = NC * NS
K = 128
NBUF = 2
ROWS_PER_TILE = 632
N_PAD = NS * ROWS_PER_TILE
_PUB_SIZES = [128] * 4 + [120]
DUMMY_ROW = N


def _sc_agg_body(ch_per_tile, h_hbm, src_hbm, dst_hbm, *refs):
    out_hbm, sidx_all, dbuf, rows, isem, acc, dsem, gsem = refs
    c = lax.axis_index("c")
    s = lax.axis_index("s")
    wid = c * NS + s
    ept = ch_per_tile * K

    idx_src_cp = pltpu.async_copy(
        src_hbm.at[pl.ds(wid * ept, ept)], sidx_all, isem)

    def dst_cp(ch, b):
        off = pl.multiple_of(wid * ept + ch * K, K)
        return pltpu.async_copy(dst_hbm.at[pl.ds(off, K)], dbuf[b], dsem[b])

    def gather_cp(ch, b):
        off = pl.multiple_of(ch * K, K)
        return pltpu.async_copy(
            h_hbm.at[sidx_all.at[pl.ds(off, K)]], rows[b], gsem[b])

    def gather_wait(b):
        pltpu.make_async_copy(
            h_hbm.at[sidx_all.at[pl.ds(0, K)]], rows[b], gsem[b]).wait()

    zv = jnp.zeros((16,), jnp.float32)

    @pl.loop(0, K)
    def _(i):
        for j in range(D // 16):
            rows[0][i, pl.ds(j * 16, 16)] = zv

    r0 = s * ROWS_PER_TILE
    for sz in _PUB_SIZES:
        pltpu.sync_copy(rows[0].at[pl.ds(0, sz)], acc.at[pl.ds(r0, sz)])
        r0 += sz

    idx_src_cp.wait()
    plsc.subcore_barrier()

    for b in range(NBUF):
        dst_cp(b, b)
        gather_cp(b, b)

    @pl.loop(0, ch_per_tile, step=NBUF)
    def _(i):
        for b in range(NBUF):
            ch = i + b
            gather_wait(b)
            pltpu.make_async_copy(
                dst_hbm.at[pl.ds(0, K)], dbuf[b], dsem[b]).wait()
            pltpu.sync_copy(rows[b], acc.at[dbuf[b]], add=True)
            nxt = ch + NBUF

            @pl.when(nxt < ch_per_tile)
            def _():
                dst_cp(nxt, b)
                gather_cp(nxt, b)

    plsc.subcore_barrier()

    r0 = s * ROWS_PER_TILE
    for sz in _PUB_SIZES:
        pltpu.sync_copy(acc.at[pl.ds(r0, sz)], rows[0].at[pl.ds(0, sz)])
        pltpu.sync_copy(rows[0].at[pl.ds(0, sz)], out_hbm.at[c, pl.ds(r0, sz)])
        r0 += sz


def _sc_deg_body(ch_per_tile, dst_hbm, deg_hbm, didx, ones_b, zbuf,
                 degacc, sem):
    c = lax.axis_index("c")
    s = lax.axis_index("s")
    wid = c * NS + s
    ept = ch_per_tile * K

    zv = jnp.zeros((16,), jnp.float32)
    ov = jnp.ones((16,), jnp.float32)

    @pl.loop(0, K)
    def _(i):
        zbuf[i, pl.ds(0, 16)] = zv
        ones_b[i, pl.ds(0, 16)] = ov

    r0 = s * ROWS_PER_TILE
    for sz in _PUB_SIZES:
        pltpu.sync_copy(zbuf.at[pl.ds(0, sz)], degacc.at[pl.ds(r0, sz)])
        r0 += sz

    plsc.subcore_barrier()

    @pl.loop(0, ch_per_tile)
    def _(i):
        off = pl.multiple_of(wid * ept + i * K, K)
        pltpu.sync_copy(dst_hbm.at[pl.ds(off, K)], didx)
        pltpu.sync_copy(ones_b, degacc.at[didx], add=True)

    plsc.subcore_barrier()

    r0 = s * ROWS_PER_TILE
    for sz in _PUB_SIZES:
        pltpu.sync_copy(degacc.at[pl.ds(r0, sz)], zbuf.at[pl.ds(0, sz)])
        pltpu.sync_copy(zbuf.at[pl.ds(0, sz)], deg_hbm.at[c, pl.ds(r0, sz)])
        r0 += sz


def _make_sc_agg(ch_per_tile):
    mesh = plsc.VectorSubcoreMesh(core_axis_name="c", subcore_axis_name="s")
    return pl.kernel(
        functools.partial(_sc_agg_body, ch_per_tile),
        out_type=jax.ShapeDtypeStruct((NC, N_PAD, D), jnp.float32),
        mesh=mesh,
        scratch_types=(
            pltpu.VMEM((ch_per_tile * K,), jnp.int32),
            tuple(pltpu.VMEM((K,), jnp.int32) for _ in range(NBUF)),
            tuple(pltpu.VMEM((K, D), jnp.float32) for _ in range(NBUF)),
            pltpu.SemaphoreType.DMA,
            pltpu.VMEM_SHARED((N_PAD, D), jnp.float32),
            tuple(pltpu.SemaphoreType.DMA for _ in range(NBUF)),
            tuple(pltpu.SemaphoreType.DMA for _ in range(NBUF)),
        ),
    )


def _make_sc_deg(ch_per_tile):
    mesh = plsc.VectorSubcoreMesh(core_axis_name="c", subcore_axis_name="s")
    return pl.kernel(
        functools.partial(_sc_deg_body, ch_per_tile),
        out_type=jax.ShapeDtypeStruct((NC, N_PAD, 16), jnp.float32),
        mesh=mesh,
        scratch_types=(
            pltpu.VMEM((K,), jnp.int32),
            pltpu.VMEM((K, 16), jnp.float32),
            pltpu.VMEM((K, 16), jnp.float32),
            pltpu.VMEM_SHARED((N_PAD, 16), jnp.float32),
            pltpu.SemaphoreType.DMA,
        ),
    )


def _tc_mean_layer_body(p0, p1, d0, d1, h, wl, bl, wr, out):
    deg = d0[:, 0:1] + d1[:, 0:1]
    inv = 1.0 / jnp.maximum(deg, 1.0)
    agg = (p0[...] + p1[...]) * inv
    y = (jnp.dot(agg, wl[...], preferred_element_type=jnp.float32)
         + bl[...]
         + jnp.dot(h[...], wr[...], preferred_element_type=jnp.float32))
    out[...] = jnp.maximum(y, 0.0)


def _tc_final_layer_body(p0, p1, h, wl, bl, wr, wlin, blin, out):
    agg = p0[...] + p1[...]
    y = (jnp.dot(agg, wl[...], preferred_element_type=jnp.float32)
         + bl[...]
         + jnp.dot(h[...], wr[...], preferred_element_type=jnp.float32))
    hh = jnp.maximum(y, 0.0)
    o = jnp.dot(hh, wlin[...], preferred_element_type=jnp.float32) + blin[...]
    e = jnp.exp(o - jnp.max(o, axis=1, keepdims=True))
    out[...] = e / jnp.sum(e, axis=1, keepdims=True)


_BM = 1264


def _row_spec(width):
    return pl.BlockSpec((_BM, width), lambda i: (i, 0))


def _full_spec(r, ccol):
    return pl.BlockSpec((r, ccol), lambda i: (0, 0))


def _tc_mean_layer(p0, p1, d0, d1, h, wl, bl, wr):
    return pl.pallas_call(
        _tc_mean_layer_body,
        grid=(N_PAD // _BM,),
        in_specs=[
            _row_spec(D), _row_spec(D), _row_spec(16), _row_spec(16),
            _row_spec(D), _full_spec(D, D), _full_spec(1, D), _full_spec(D, D),
        ],
        out_specs=_row_spec(D),
        out_shape=jax.ShapeDtypeStruct((N_PAD, D), jnp.float32),
    )(p0, p1, d0, d1, h, wl, bl.reshape(1, D), wr)


def _tc_final_layer(p0, p1, h, wl, bl, wr, wlin, blin):
    return pl.pallas_call(
        _tc_final_layer_body,
        grid=(N_PAD // _BM,),
        in_specs=[
            _row_spec(D), _row_spec(D), _row_spec(D),
            _full_spec(D, D), _full_spec(1, D), _full_spec(D, D),
            _full_spec(D, 1), _full_spec(1, 1),
        ],
        out_specs=_row_spec(1),
        out_shape=jax.ShapeDtypeStruct((N_PAD, 1), jnp.float32),
    )(p0, p1, h, wl, bl.reshape(1, D), wr, wlin, blin.reshape(1, 1))


def kernel(x, edge_index, W1l, b1l, W1r, W2l, b2l, W2r, W3l, b3l, W3r,
           Wlin, blin):
    e = edge_index.shape[1]
    ch_per_tile = -(-e // (NW * K))
    ch_per_tile = -(-ch_per_tile // NBUF) * NBUF
    e_pad = ch_per_tile * NW * K
    src = edge_index[0].astype(jnp.int32)
    dst = edge_index[1].astype(jnp.int32)
    pad = e_pad - e
    if pad:
        fill = jnp.arange(pad, dtype=jnp.int32)
        src = jnp.concatenate([src, fill % N])
        dst = jnp.concatenate([dst, DUMMY_ROW + fill % (N_PAD - N)])
    xp = jnp.concatenate([x, jnp.zeros((N_PAD - N, D), x.dtype)])

    sc_agg = _make_sc_agg(ch_per_tile)
    sc_deg = _make_sc_deg(ch_per_tile)

    def _one(r):
        return r[0] if isinstance(r, (tuple, list)) else r

    dp = _one(sc_deg(dst))
    p = _one(sc_agg(xp, src, dst))
    h1 = _tc_mean_layer(p[0], p[1], dp[0], dp[1], xp, W1l, b1l, W1r)
    p = _one(sc_agg(h1, src, dst))
    h2 = _tc_mean_layer(p[0], p[1], dp[0], dp[1], h1, W2l, b2l, W2r)
    p = _one(sc_agg(h2, src, dst))
    out = _tc_final_layer(p[0], p[1], h2, W3l, b3l, W3r, Wlin, blin)
    return out[:N]


# speedup vs baseline: 6.3844x; 1.0505x over previous
import functools

import jax
import jax.numpy as jnp
from jax import lax
from jax.experimental import pallas as pl
from jax.experimental.pallas import tpu as pltpu
from jax.experimental.pallas import tpu_sc as plsc

N = 10000
D = 128
NC = 2
NS = 16
NW = NC * NS
K = 128
NBUF = 2
ROWS_PER_TILE = 632
N_PAD = NS * ROWS_PER_TILE
_PUB_SIZES = [128] * 4 + [120]
DUMMY_ROW = N


def _sc_agg_body(ch_per_tile, h_hbm, src_hbm, dst_hbm, *refs):
    out_hbm, sidx_all, dbuf, rows, isem, acc, dsem, gsem = refs
    c = lax.axis_index("c")
    s = lax.axis_index("s")
    wid = c * NS + s
    ept = ch_per_tile * K

    idx_src_cp = pltpu.async_copy(
        src_hbm.at[pl.ds(wid * ept, ept)], sidx_all, isem)

    def dst_cp(ch, b):
        off = pl.multiple_of(wid * ept + ch * K, K)
        return pltpu.async_copy(dst_hbm.at[pl.ds(off, K)], dbuf[b], dsem[b])

    def gather_cp(ch, b):
        off = pl.multiple_of(ch * K, K)
        return pltpu.async_copy(
            h_hbm.at[sidx_all.at[pl.ds(off, K)]], rows[b], gsem[b])

    def gather_wait(b):
        pltpu.make_async_copy(
            h_hbm.at[sidx_all.at[pl.ds(0, K)]], rows[b], gsem[b]).wait()

    zv = jnp.zeros((16,), jnp.float32)

    @pl.loop(0, K)
    def _(i):
        for j in range(D // 16):
            rows[0][i, pl.ds(j * 16, 16)] = zv

    r0 = s * ROWS_PER_TILE
    for sz in _PUB_SIZES:
        pltpu.sync_copy(rows[0].at[pl.ds(0, sz)], acc.at[pl.ds(r0, sz)])
        r0 += sz

    idx_src_cp.wait()
    plsc.subcore_barrier()

    for b in range(NBUF):
        dst_cp(b, b)
        gather_cp(b, b)

    @pl.loop(0, ch_per_tile, step=NBUF)
    def _(i):
        for b in range(NBUF):
            ch = i + b
            gather_wait(b)
            pltpu.make_async_copy(
                dst_hbm.at[pl.ds(0, K)], dbuf[b], dsem[b]).wait()
            pltpu.sync_copy(rows[b], acc.at[dbuf[b]], add=True)
            nxt = ch + NBUF

            @pl.when(nxt < ch_per_tile)
            def _():
                dst_cp(nxt, b)
                gather_cp(nxt, b)

    plsc.subcore_barrier()

    r0 = s * ROWS_PER_TILE
    for sz in _PUB_SIZES:
        pltpu.sync_copy(acc.at[pl.ds(r0, sz)], rows[0].at[pl.ds(0, sz)])
        pltpu.sync_copy(rows[0].at[pl.ds(0, sz)], out_hbm.at[c, pl.ds(r0, sz)])
        r0 += sz


def _sc_deg_body(ch_per_tile, dst_hbm, deg_hbm, didx, ones_b, zbuf,
                 degacc, sem):
    c = lax.axis_index("c")
    s = lax.axis_index("s")
    wid = c * NS + s
    ept = ch_per_tile * K

    zv = jnp.zeros((16,), jnp.float32)
    ov = jnp.ones((16,), jnp.float32)

    @pl.loop(0, K)
    def _(i):
        zbuf[i, pl.ds(0, 16)] = zv
        ones_b[i, pl.ds(0, 16)] = ov

    r0 = s * ROWS_PER_TILE
    for sz in _PUB_SIZES:
        pltpu.sync_copy(zbuf.at[pl.ds(0, sz)], degacc.at[pl.ds(r0, sz)])
        r0 += sz

    plsc.subcore_barrier()

    @pl.loop(0, ch_per_tile)
    def _(i):
        off = pl.multiple_of(wid * ept + i * K, K)
        pltpu.sync_copy(dst_hbm.at[pl.ds(off, K)], didx)
        pltpu.sync_copy(ones_b, degacc.at[didx], add=True)

    plsc.subcore_barrier()

    r0 = s * ROWS_PER_TILE
    for sz in _PUB_SIZES:
        pltpu.sync_copy(degacc.at[pl.ds(r0, sz)], zbuf.at[pl.ds(0, sz)])
        pltpu.sync_copy(zbuf.at[pl.ds(0, sz)], deg_hbm.at[c, pl.ds(r0, sz)])
        r0 += sz


def _make_sc_agg(ch_per_tile):
    mesh = plsc.VectorSubcoreMesh(core_axis_name="c", subcore_axis_name="s")
    return pl.kernel(
        functools.partial(_sc_agg_body, ch_per_tile),
        out_type=jax.ShapeDtypeStruct((NC, N_PAD, D), jnp.float32),
        mesh=mesh,
        scratch_types=(
            pltpu.VMEM((ch_per_tile * K,), jnp.int32),
            tuple(pltpu.VMEM((K,), jnp.int32) for _ in range(NBUF)),
            tuple(pltpu.VMEM((K, D), jnp.float32) for _ in range(NBUF)),
            pltpu.SemaphoreType.DMA,
            pltpu.VMEM_SHARED((N_PAD, D), jnp.float32),
            tuple(pltpu.SemaphoreType.DMA for _ in range(NBUF)),
            tuple(pltpu.SemaphoreType.DMA for _ in range(NBUF)),
        ),
    )


def _make_sc_deg(ch_per_tile):
    mesh = plsc.VectorSubcoreMesh(core_axis_name="c", subcore_axis_name="s")
    return pl.kernel(
        functools.partial(_sc_deg_body, ch_per_tile),
        out_type=jax.ShapeDtypeStruct((NC, N_PAD, 16), jnp.float32),
        mesh=mesh,
        scratch_types=(
            pltpu.VMEM((K,), jnp.int32),
            pltpu.VMEM((K, 16), jnp.float32),
            pltpu.VMEM((K, 16), jnp.float32),
            pltpu.VMEM_SHARED((N_PAD, 16), jnp.float32),
            pltpu.SemaphoreType.DMA,
        ),
    )


def _tc_mean_layer_body(p0, p1, d0, d1, h, wl, bl, wr, out):
    deg = d0[0, :, 0:1] + d1[0, :, 0:1]
    inv = 1.0 / jnp.maximum(deg, 1.0)
    agg = (p0[0] + p1[0]) * inv
    y = (jnp.dot(agg, wl[...], preferred_element_type=jnp.float32)
         + bl[...]
         + jnp.dot(h[...], wr[...], preferred_element_type=jnp.float32))
    out[...] = jnp.maximum(y, 0.0)


def _tc_final_layer_body(p0, p1, h, wl, bl, wr, wlin, blin, out):
    agg = p0[0] + p1[0]
    y = (jnp.dot(agg, wl[...], preferred_element_type=jnp.float32)
         + bl[...]
         + jnp.dot(h[...], wr[...], preferred_element_type=jnp.float32))
    hh = jnp.maximum(y, 0.0)
    o = jnp.dot(hh, wlin[...], preferred_element_type=jnp.float32) + blin[...]
    e = jnp.exp(o - jnp.max(o, axis=1, keepdims=True))
    out[...] = e / jnp.sum(e, axis=1, keepdims=True)


_BM = 1264


def _part_spec(width, part):
    return pl.BlockSpec((1, _BM, width), lambda i, _p=part: (_p, i, 0))


def _row_spec(width):
    return pl.BlockSpec((_BM, width), lambda i: (i, 0))


def _full_spec(r, ccol):
    return pl.BlockSpec((r, ccol), lambda i: (0, 0))


def _tc_mean_layer(p, dp, h, wl, bl, wr):
    return pl.pallas_call(
        _tc_mean_layer_body,
        grid=(N_PAD // _BM,),
        in_specs=[
            _part_spec(D, 0), _part_spec(D, 1),
            _part_spec(16, 0), _part_spec(16, 1),
            _row_spec(D), _full_spec(D, D), _full_spec(1, D), _full_spec(D, D),
        ],
        out_specs=_row_spec(D),
        out_shape=jax.ShapeDtypeStruct((N_PAD, D), jnp.float32),
    )(p, p, dp, dp, h, wl, bl.reshape(1, D), wr)


def _tc_final_layer(p, h, wl, bl, wr, wlin, blin):
    return pl.pallas_call(
        _tc_final_layer_body,
        grid=(N_PAD // _BM,),
        in_specs=[
            _part_spec(D, 0), _part_spec(D, 1), _row_spec(D),
            _full_spec(D, D), _full_spec(1, D), _full_spec(D, D),
            _full_spec(D, 1), _full_spec(1, 1),
        ],
        out_specs=_row_spec(1),
        out_shape=jax.ShapeDtypeStruct((N_PAD, 1), jnp.float32),
    )(p, p, h, wl, bl.reshape(1, D), wr, wlin, blin.reshape(1, 1))


def kernel(x, edge_index, W1l, b1l, W1r, W2l, b2l, W2r, W3l, b3l, W3r,
           Wlin, blin):
    e = edge_index.shape[1]
    ch_per_tile = -(-e // (NW * K))
    ch_per_tile = -(-ch_per_tile // NBUF) * NBUF
    e_pad = ch_per_tile * NW * K
    src = edge_index[0].astype(jnp.int32)
    dst = edge_index[1].astype(jnp.int32)
    pad = e_pad - e
    if pad:
        fill = jnp.arange(pad, dtype=jnp.int32)
        src = jnp.concatenate([src, fill % N])
        dst = jnp.concatenate([dst, DUMMY_ROW + fill % (N_PAD - N)])
    xp = jnp.concatenate([x, jnp.zeros((N_PAD - N, D), x.dtype)])

    sc_agg = _make_sc_agg(ch_per_tile)
    sc_deg = _make_sc_deg(ch_per_tile)

    def _one(r):
        return r[0] if isinstance(r, (tuple, list)) else r

    dp = _one(sc_deg(dst))
    p = _one(sc_agg(xp, src, dst))
    h1 = _tc_mean_layer(p, dp, xp, W1l, b1l, W1r)
    p = _one(sc_agg(h1, src, dst))
    h2 = _tc_mean_layer(p, dp, h1, W2l, b2l, W2r)
    p = _one(sc_agg(h2, src, dst))
    out = _tc_final_layer(p, h2, W3l, b3l, W3r, Wlin, blin)
    return out[:N]
